# Initial kernel scaffold; baseline (speedup 1.0000x reference)
import jax
import jax.numpy as jnp
from jax.experimental import pallas as pl


def kernel(x, edge_index, Wp, bp, W1, as1, ad1, b1, W2, as2, ad2, b2):
    raise NotImplementedError("write your pallas kernel here")


# speedup vs baseline: 24.0600x; 24.0600x over previous
import functools

import jax
import jax.numpy as jnp
from jax import lax
from jax.experimental import pallas as pl
from jax.experimental.pallas import tpu as pltpu
from jax.experimental.pallas import tpu_sc as plsc

_N = 10000
_D = 128
_E = 320000
_NC = 2
_NS = 16
_NW = _NC * _NS
_ET = _E // _NW
_CH = 80
_NCH = _ET // _CH
_STRIPE = 632
_NP = _STRIPE * _NS
_BLK = 128
_NB = 79
_NPAD = _NB * _BLK


_GRP = 25
_NG = _NCH // _GRP


def _wpass_kernel_fn():
    mesh = plsc.VectorSubcoreMesh(core_axis_name="c", subcore_axis_name="s")

    @functools.partial(
        pl.kernel,
        out_type=(
            jax.ShapeDtypeStruct((_NW, _NCH, _CH), jnp.float32),
            jax.ShapeDtypeStruct((_NW * _NPAD,), jnp.float32),
        ),
        mesh=mesh,
        scratch_types=[
            pltpu.VMEM((_NCH, _CH), jnp.int32),
            pltpu.VMEM((_NCH, _CH), jnp.int32),
            pltpu.VMEM((_N,), jnp.float32),
            pltpu.VMEM((_N,), jnp.float32),
            pltpu.VMEM((_N,), jnp.float32),
            pltpu.VMEM((_NCH, _CH), jnp.float32),
        ],
        compiler_params=pltpu.CompilerParams(needs_layout_passes=False),
    )
    def wpass(asrc_hbm, adst_hbm, src_hbm, dst_hbm,
              w_hbm, dacc_hbm,
              src_v, dst_v, asrc_v, adst_v, dacc_v, w_v):
        cid = lax.axis_index("c")
        sid = lax.axis_index("s")
        wid = sid * _NC + cid

        pltpu.sync_copy(src_hbm.at[wid], src_v)
        pltpu.sync_copy(dst_hbm.at[wid], dst_v)
        pltpu.sync_copy(asrc_hbm, asrc_v)
        pltpu.sync_copy(adst_hbm, adst_v)

        def zloop(i, carry):
            dacc_v[pl.ds(i * 16, 16)] = jnp.zeros((16,), jnp.float32)
            return carry

        lax.fori_loop(0, _N // 16, zloop, 0)

        def chunk(j, carry):
            def wloop(k, c2):
                sv = src_v[j, pl.ds(k * 16, 16)]
                dv = dst_v[j, pl.ds(k * 16, 16)]
                a = plsc.load_gather(asrc_v, [sv])
                b = plsc.load_gather(adst_v, [dv])
                s = a + b
                w = jnp.exp(jnp.where(s >= 0, s, s * 0.2))
                w_v[j, pl.ds(k * 16, 16)] = w
                plsc.addupdate_scatter(dacc_v, [dv], w)
                return c2

            lax.fori_loop(0, _CH // 16, wloop, 0)
            return carry

        lax.fori_loop(0, _NCH, chunk, 0)

        pltpu.sync_copy(w_v, w_hbm.at[wid])
        pltpu.sync_copy(dacc_v, dacc_hbm.at[pl.ds(wid * _NPAD, _N)])

    return wpass


def _rowpass_kernel_fn():
    mesh = plsc.VectorSubcoreMesh(core_axis_name="c", subcore_axis_name="s")

    @functools.partial(
        pl.kernel,
        out_type=jax.ShapeDtypeStruct((_NC, _NP, _D), jnp.float32),
        mesh=mesh,
        scratch_types=[
            pltpu.VMEM((_GRP, _CH), jnp.int32),
            pltpu.VMEM((_GRP, _CH), jnp.int32),
            pltpu.VMEM((_GRP, _CH), jnp.float32),
            pltpu.VMEM((_CH, _D), jnp.float32),
            pltpu.VMEM_SHARED((_NP, _D), jnp.float32),
            pltpu.SemaphoreType.DMA,
        ],
        compiler_params=pltpu.CompilerParams(needs_layout_passes=False),
    )
    def rowpass(h_hbm, src_hbm, dst_hbm, w_hbm, z_hbm,
                accp_hbm,
                src_v, dst_v, w_v, rowbuf, acc_sh, sem):
        cid = lax.axis_index("c")
        sid = lax.axis_index("s")
        wid = sid * _NC + cid

        rbase = sid * _STRIPE
        pltpu.sync_copy(z_hbm.at[pl.ds(rbase, _STRIPE)],
                        acc_sh.at[pl.ds(rbase, _STRIPE)])
        plsc.subcore_barrier()

        def group(g, carry):
            pltpu.sync_copy(src_hbm.at[wid, g], src_v)
            pltpu.sync_copy(dst_hbm.at[wid, g], dst_v)
            pltpu.sync_copy(w_hbm.at[wid, g], w_v)

            def chunk(j, c2):
                pltpu.async_copy(h_hbm.at[src_v.at[j]], rowbuf, sem).wait()
                j16 = jnp.zeros((16,), jnp.int32) + j

                def scale(r, c3):
                    wv = plsc.load_gather(
                        w_v, [j16, jnp.zeros((16,), jnp.int32) + r])
                    for c in range(_D // 16):
                        rowbuf[r, pl.ds(c * 16, 16)] = (
                            rowbuf[r, pl.ds(c * 16, 16)] * wv)
                    return c3

                lax.fori_loop(0, _CH, scale, 0)
                pltpu.sync_copy(rowbuf, acc_sh.at[dst_v.at[j]], add=True)
                return c2

            lax.fori_loop(0, _GRP, chunk, 0)
            return carry

        lax.fori_loop(0, _NG, group, 0)
        plsc.subcore_barrier()

        pltpu.sync_copy(acc_sh.at[pl.ds(rbase, _STRIPE)],
                        accp_hbm.at[cid, pl.ds(rbase, _STRIPE)])

    return rowpass


_wpass = _wpass_kernel_fn()
_rowpass = _rowpass_kernel_fn()


def _edge(h, asrc, adst, src, dst, z):
    w, dacc = _wpass(asrc, adst, src, dst)
    accp = _rowpass(h,
                    src.reshape(_NW, _NG, _GRP, _CH),
                    dst.reshape(_NW, _NG, _GRP, _CH),
                    w.reshape(_NW, _NG, _GRP, _CH), z)
    return accp, dacc


def _full_spec():
    return pl.BlockSpec((_D, _D), lambda i: (0, 0))


def _vec_spec():
    return pl.BlockSpec((_D,), lambda i: (0,))


def _row_spec():
    return pl.BlockSpec((_BLK, _D), lambda i: (i, 0))


def _n_spec():
    return pl.BlockSpec((_NPAD,), lambda i: (0,))


def _prep1(x, Wp, bp, W1, as1, ad1):
    def body(x_r, Wp_r, bp_r, W1_r, as1_r, ad1_r, h1_r, s_r, d_r):
        i = pl.program_id(0)
        h0 = jnp.dot(x_r[...], Wp_r[...],
                     preferred_element_type=jnp.float32) + bp_r[...][None, :]
        h1 = jnp.dot(h0, W1_r[...], preferred_element_type=jnp.float32)
        h1_r[...] = h1
        s_r[pl.ds(i * _BLK, _BLK)] = jnp.sum(h1 * as1_r[...][None, :], axis=1)
        d_r[pl.ds(i * _BLK, _BLK)] = jnp.sum(h1 * ad1_r[...][None, :], axis=1)

    return pl.pallas_call(
        body,
        grid=(_NB,),
        in_specs=[_row_spec(), _full_spec(), _vec_spec(), _full_spec(),
                  _vec_spec(), _vec_spec()],
        out_specs=[_row_spec(), _n_spec(), _n_spec()],
        out_shape=[
            jax.ShapeDtypeStruct((_N, _D), jnp.float32),
            jax.ShapeDtypeStruct((_NPAD,), jnp.float32),
            jax.ShapeDtypeStruct((_NPAD,), jnp.float32),
        ],
    )(x, Wp, bp, W1, as1, ad1)


def _dacc_spec():
    return pl.BlockSpec((_NW, _NPAD), lambda i: (0, 0))


def _combine_mid(acc0, acc1, dacc, asrc, adst, h1, b1, W2, as2, ad2):
    def body(a0, a1, dc, sr, dr, h1r, b1r, W2r, as2r, ad2r,
             h2_r, s2_r, d2_r):
        i = pl.program_id(0)
        blk = pl.ds(i * _BLK, _BLK)
        s = sr[blk] + dr[blk]
        wl = jnp.exp(jnp.where(s >= 0, s, s * 0.2))
        num = a0[...] + a1[...] + wl[:, None] * h1r[...]
        den = jnp.sum(dc[:, blk], axis=0) + wl + 1e-16
        g = jnp.maximum(num / den[:, None] + b1r[...][None, :], 0.0)
        h2 = jnp.dot(g, W2r[...], preferred_element_type=jnp.float32)
        h2_r[...] = h2
        s2_r[blk] = jnp.sum(h2 * as2r[...][None, :], axis=1)
        d2_r[blk] = jnp.sum(h2 * ad2r[...][None, :], axis=1)

    return pl.pallas_call(
        body,
        grid=(_NB,),
        in_specs=[_row_spec(), _row_spec(), _dacc_spec(), _n_spec(),
                  _n_spec(), _row_spec(), _vec_spec(), _full_spec(),
                  _vec_spec(), _vec_spec()],
        out_specs=[_row_spec(), _n_spec(), _n_spec()],
        out_shape=[
            jax.ShapeDtypeStruct((_N, _D), jnp.float32),
            jax.ShapeDtypeStruct((_NPAD,), jnp.float32),
            jax.ShapeDtypeStruct((_NPAD,), jnp.float32),
        ],
    )(acc0, acc1, dacc, asrc, adst, h1, b1, W2, as2, ad2)


def _combine_final(acc0, acc1, dacc, asrc, adst, h2, b2):
    def body(a0, a1, dc, sr, dr, h2r, b2r, out_r):
        i = pl.program_id(0)
        blk = pl.ds(i * _BLK, _BLK)
        s = sr[blk] + dr[blk]
        wl = jnp.exp(jnp.where(s >= 0, s, s * 0.2))
        num = a0[...] + a1[...] + wl[:, None] * h2r[...]
        den = jnp.sum(dc[:, blk], axis=0) + wl + 1e-16
        out_r[...] = num / den[:, None] + b2r[...][None, :]

    return pl.pallas_call(
        body,
        grid=(_NB,),
        in_specs=[_row_spec(), _row_spec(), _dacc_spec(), _n_spec(),
                  _n_spec(), _row_spec(), _vec_spec()],
        out_specs=_row_spec(),
        out_shape=jax.ShapeDtypeStruct((_N, _D), jnp.float32),
    )(acc0, acc1, dacc, asrc, adst, h2, b2)


def kernel(x, edge_index, Wp, bp, W1, as1, ad1, b1, W2, as2, ad2, b2):
    src = edge_index[0].reshape(_NW, _NCH, _CH)
    dst = edge_index[1].reshape(_NW, _NCH, _CH)
    z = jnp.zeros((_NP, _D), jnp.float32)

    h1, s1, d1 = _prep1(x, Wp, bp, W1, as1, ad1)
    accp1, dacc1 = _edge(h1, s1[:_N], d1[:_N], src, dst, z)
    h2, s2, d2 = _combine_mid(accp1[0], accp1[1], dacc1.reshape(_NW, _NPAD),
                              s1, d1, h1, b1, W2, as2, ad2)
    accp2, dacc2 = _edge(h2, s2[:_N], d2[:_N], src, dst, z)
    return _combine_final(accp2[0], accp2[1], dacc2.reshape(_NW, _NPAD),
                          s2, d2, h2, b2)


# speedup vs baseline: 33.4818x; 1.3916x over previous
import functools

import jax
import jax.numpy as jnp
from jax import lax
from jax.experimental import pallas as pl
from jax.experimental.pallas import tpu as pltpu
from jax.experimental.pallas import tpu_sc as plsc

_N = 10000
_D = 128
_E = 320000
_NC = 2
_NS = 16
_NW = _NC * _NS
_ET = _E // _NW
_CH = 80
_NCH = _ET // _CH
_STRIPE = 632
_NP = _STRIPE * _NS
_BLK = 128
_NB = 79
_NPAD = _NB * _BLK


_GRP = 25
_NG = _NCH // _GRP


def _wpass_kernel_fn():
    mesh = plsc.VectorSubcoreMesh(core_axis_name="c", subcore_axis_name="s")

    @functools.partial(
        pl.kernel,
        out_type=(
            jax.ShapeDtypeStruct((_NW, _NCH, _CH), jnp.float32),
            jax.ShapeDtypeStruct((_NW * _NPAD,), jnp.float32),
        ),
        mesh=mesh,
        scratch_types=[
            pltpu.VMEM((_NCH, _CH), jnp.int32),
            pltpu.VMEM((_NCH, _CH), jnp.int32),
            pltpu.VMEM((_N,), jnp.float32),
            pltpu.VMEM((_N,), jnp.float32),
            pltpu.VMEM((_N,), jnp.float32),
            pltpu.VMEM((_NCH, _CH), jnp.float32),
        ],
        compiler_params=pltpu.CompilerParams(needs_layout_passes=False),
    )
    def wpass(asrc_hbm, adst_hbm, src_hbm, dst_hbm,
              w_hbm, dacc_hbm,
              src_v, dst_v, asrc_v, adst_v, dacc_v, w_v):
        cid = lax.axis_index("c")
        sid = lax.axis_index("s")
        wid = sid * _NC + cid

        pltpu.sync_copy(src_hbm.at[wid], src_v)
        pltpu.sync_copy(dst_hbm.at[wid], dst_v)
        pltpu.sync_copy(asrc_hbm, asrc_v)
        pltpu.sync_copy(adst_hbm, adst_v)

        def zloop(i, carry):
            dacc_v[pl.ds(i * 16, 16)] = jnp.zeros((16,), jnp.float32)
            return carry

        lax.fori_loop(0, _N // 16, zloop, 0)

        def chunk(j, carry):
            def wloop(k, c2):
                sv = src_v[j, pl.ds(k * 16, 16)]
                dv = dst_v[j, pl.ds(k * 16, 16)]
                a = plsc.load_gather(asrc_v, [sv])
                b = plsc.load_gather(adst_v, [dv])
                s = a + b
                w = jnp.exp(jnp.where(s >= 0, s, s * 0.2))
                w_v[j, pl.ds(k * 16, 16)] = w
                plsc.addupdate_scatter(dacc_v, [dv], w)
                return c2

            lax.fori_loop(0, _CH // 16, wloop, 0)
            return carry

        lax.fori_loop(0, _NCH, chunk, 0)

        pltpu.sync_copy(w_v, w_hbm.at[wid])
        pltpu.sync_copy(dacc_v, dacc_hbm.at[pl.ds(wid * _NPAD, _N)])

    return wpass


def _rowpass_kernel_fn():
    mesh = plsc.VectorSubcoreMesh(core_axis_name="c", subcore_axis_name="s")

    @functools.partial(
        pl.kernel,
        out_type=jax.ShapeDtypeStruct((_NC, _NP, _D), jnp.float32),
        mesh=mesh,
        scratch_types=[
            pltpu.VMEM((_GRP, _CH), jnp.int32),
            pltpu.VMEM((_GRP, _CH), jnp.int32),
            pltpu.VMEM((_GRP, _CH), jnp.float32),
            pltpu.VMEM((2, _CH, _D), jnp.float32),
            pltpu.VMEM_SHARED((_NP, _D), jnp.float32),
            pltpu.SemaphoreType.DMA,
        ],
        compiler_params=pltpu.CompilerParams(needs_layout_passes=False),
    )
    def rowpass(h_hbm, src_hbm, dst_hbm, w_hbm, z_hbm,
                accp_hbm,
                src_v, dst_v, w_v, rowbuf, acc_sh, sem):
        cid = lax.axis_index("c")
        sid = lax.axis_index("s")
        wid = sid * _NC + cid

        rbase = sid * _STRIPE
        pltpu.sync_copy(z_hbm.at[pl.ds(rbase, _STRIPE)],
                        acc_sh.at[pl.ds(rbase, _STRIPE)])
        plsc.subcore_barrier()

        def group(g, carry):
            pltpu.sync_copy(src_hbm.at[wid, g], src_v)
            pltpu.sync_copy(dst_hbm.at[wid, g], dst_v)
            pltpu.sync_copy(w_hbm.at[wid, g], w_v)

            pltpu.async_copy(h_hbm.at[src_v.at[0]], rowbuf.at[0], sem)

            def chunk(j, c2):
                b = lax.rem(j, 2)
                pltpu.make_async_copy(
                    h_hbm.at[src_v.at[j]], rowbuf.at[b], sem).wait()

                @pl.when(j + 1 < _GRP)
                def _():
                    pltpu.async_copy(
                        h_hbm.at[src_v.at[j + 1]], rowbuf.at[1 - b], sem)

                j16 = jnp.zeros((16,), jnp.int32) + j

                def scale(r, c3):
                    wv = plsc.load_gather(
                        w_v, [j16, jnp.zeros((16,), jnp.int32) + r])
                    for c in range(_D // 16):
                        rowbuf[b, r, pl.ds(c * 16, 16)] = (
                            rowbuf[b, r, pl.ds(c * 16, 16)] * wv)
                    return c3

                lax.fori_loop(0, _CH, scale, 0)
                pltpu.sync_copy(rowbuf.at[b], acc_sh.at[dst_v.at[j]],
                                add=True)
                return c2

            lax.fori_loop(0, _GRP, chunk, 0)
            return carry

        lax.fori_loop(0, _NG, group, 0)
        plsc.subcore_barrier()

        pltpu.sync_copy(acc_sh.at[pl.ds(rbase, _STRIPE)],
                        accp_hbm.at[cid, pl.ds(rbase, _STRIPE)])

    return rowpass


_wpass = _wpass_kernel_fn()
_rowpass = _rowpass_kernel_fn()


def _edge(h, asrc, adst, src, dst, z):
    w, dacc = _wpass(asrc, adst, src, dst)
    accp = _rowpass(h,
                    src.reshape(_NW, _NG, _GRP, _CH),
                    dst.reshape(_NW, _NG, _GRP, _CH),
                    w.reshape(_NW, _NG, _GRP, _CH), z)
    return accp, dacc


def _full_spec():
    return pl.BlockSpec((_D, _D), lambda i: (0, 0))


def _vec_spec():
    return pl.BlockSpec((_D,), lambda i: (0,))


def _row_spec():
    return pl.BlockSpec((_BLK, _D), lambda i: (i, 0))


def _n_spec():
    return pl.BlockSpec((_NPAD,), lambda i: (0,))


def _prep1(x, Wp, bp, W1, as1, ad1):
    def body(x_r, Wp_r, bp_r, W1_r, as1_r, ad1_r, h1_r, s_r, d_r):
        i = pl.program_id(0)
        h0 = jnp.dot(x_r[...], Wp_r[...],
                     preferred_element_type=jnp.float32) + bp_r[...][None, :]
        h1 = jnp.dot(h0, W1_r[...], preferred_element_type=jnp.float32)
        h1_r[...] = h1
        s_r[pl.ds(i * _BLK, _BLK)] = jnp.sum(h1 * as1_r[...][None, :], axis=1)
        d_r[pl.ds(i * _BLK, _BLK)] = jnp.sum(h1 * ad1_r[...][None, :], axis=1)

    return pl.pallas_call(
        body,
        grid=(_NB,),
        in_specs=[_row_spec(), _full_spec(), _vec_spec(), _full_spec(),
                  _vec_spec(), _vec_spec()],
        out_specs=[_row_spec(), _n_spec(), _n_spec()],
        out_shape=[
            jax.ShapeDtypeStruct((_N, _D), jnp.float32),
            jax.ShapeDtypeStruct((_NPAD,), jnp.float32),
            jax.ShapeDtypeStruct((_NPAD,), jnp.float32),
        ],
    )(x, Wp, bp, W1, as1, ad1)


def _dacc_spec():
    return pl.BlockSpec((_NW, _NPAD), lambda i: (0, 0))


def _combine_mid(acc0, acc1, dacc, asrc, adst, h1, b1, W2, as2, ad2):
    def body(a0, a1, dc, sr, dr, h1r, b1r, W2r, as2r, ad2r,
             h2_r, s2_r, d2_r):
        i = pl.program_id(0)
        blk = pl.ds(i * _BLK, _BLK)
        s = sr[blk] + dr[blk]
        wl = jnp.exp(jnp.where(s >= 0, s, s * 0.2))
        num = a0[...] + a1[...] + wl[:, None] * h1r[...]
        den = jnp.sum(dc[:, blk], axis=0) + wl + 1e-16
        g = jnp.maximum(num / den[:, None] + b1r[...][None, :], 0.0)
        h2 = jnp.dot(g, W2r[...], preferred_element_type=jnp.float32)
        h2_r[...] = h2
        s2_r[blk] = jnp.sum(h2 * as2r[...][None, :], axis=1)
        d2_r[blk] = jnp.sum(h2 * ad2r[...][None, :], axis=1)

    return pl.pallas_call(
        body,
        grid=(_NB,),
        in_specs=[_row_spec(), _row_spec(), _dacc_spec(), _n_spec(),
                  _n_spec(), _row_spec(), _vec_spec(), _full_spec(),
                  _vec_spec(), _vec_spec()],
        out_specs=[_row_spec(), _n_spec(), _n_spec()],
        out_shape=[
            jax.ShapeDtypeStruct((_N, _D), jnp.float32),
            jax.ShapeDtypeStruct((_NPAD,), jnp.float32),
            jax.ShapeDtypeStruct((_NPAD,), jnp.float32),
        ],
    )(acc0, acc1, dacc, asrc, adst, h1, b1, W2, as2, ad2)


def _combine_final(acc0, acc1, dacc, asrc, adst, h2, b2):
    def body(a0, a1, dc, sr, dr, h2r, b2r, out_r):
        i = pl.program_id(0)
        blk = pl.ds(i * _BLK, _BLK)
        s = sr[blk] + dr[blk]
        wl = jnp.exp(jnp.where(s >= 0, s, s * 0.2))
        num = a0[...] + a1[...] + wl[:, None] * h2r[...]
        den = jnp.sum(dc[:, blk], axis=0) + wl + 1e-16
        out_r[...] = num / den[:, None] + b2r[...][None, :]

    return pl.pallas_call(
        body,
        grid=(_NB,),
        in_specs=[_row_spec(), _row_spec(), _dacc_spec(), _n_spec(),
                  _n_spec(), _row_spec(), _vec_spec()],
        out_specs=_row_spec(),
        out_shape=jax.ShapeDtypeStruct((_N, _D), jnp.float32),
    )(acc0, acc1, dacc, asrc, adst, h2, b2)


def kernel(x, edge_index, Wp, bp, W1, as1, ad1, b1, W2, as2, ad2, b2):
    src = edge_index[0].reshape(_NW, _NCH, _CH)
    dst = edge_index[1].reshape(_NW, _NCH, _CH)
    z = jnp.zeros((_NP, _D), jnp.float32)

    h1, s1, d1 = _prep1(x, Wp, bp, W1, as1, ad1)
    accp1, dacc1 = _edge(h1, s1[:_N], d1[:_N], src, dst, z)
    h2, s2, d2 = _combine_mid(accp1[0], accp1[1], dacc1.reshape(_NW, _NPAD),
                              s1, d1, h1, b1, W2, as2, ad2)
    accp2, dacc2 = _edge(h2, s2[:_N], d2[:_N], src, dst, z)
    return _combine_final(accp2[0], accp2[1], dacc2.reshape(_NW, _NPAD),
                          s2, d2, h2, b2)


# speedup vs baseline: 33.5090x; 1.0008x over previous
import functools

import jax
import jax.numpy as jnp
from jax import lax
from jax.experimental import pallas as pl
from jax.experimental.pallas import tpu as pltpu
from jax.experimental.pallas import tpu_sc as plsc

_N = 10000
_D = 128
_E = 320000
_NC = 2
_NS = 16
_NW = _NC * _NS
_ET = _E // _NW
_CH = 80
_NCH = _ET // _CH
_STRIPE = 632
_NP = _STRIPE * _NS
_BLK = 128
_NB = 79
_NPAD = _NB * _BLK


_GRP = 25
_NG = _NCH // _GRP


def _wpass_kernel_fn():
    mesh = plsc.VectorSubcoreMesh(core_axis_name="c", subcore_axis_name="s")

    @functools.partial(
        pl.kernel,
        out_type=(
            jax.ShapeDtypeStruct((_NW, _NCH, _CH), jnp.float32),
            jax.ShapeDtypeStruct((_NW * _NPAD,), jnp.float32),
        ),
        mesh=mesh,
        scratch_types=[
            pltpu.VMEM((_NCH, _CH), jnp.int32),
            pltpu.VMEM((_NCH, _CH), jnp.int32),
            pltpu.VMEM((_N,), jnp.float32),
            pltpu.VMEM((_N,), jnp.float32),
            pltpu.VMEM((_N,), jnp.float32),
            pltpu.VMEM((_NCH, _CH), jnp.float32),
        ],
        compiler_params=pltpu.CompilerParams(needs_layout_passes=False),
    )
    def wpass(asrc_hbm, adst_hbm, src_hbm, dst_hbm,
              w_hbm, dacc_hbm,
              src_v, dst_v, asrc_v, adst_v, dacc_v, w_v):
        cid = lax.axis_index("c")
        sid = lax.axis_index("s")
        wid = sid * _NC + cid

        pltpu.sync_copy(src_hbm.at[wid], src_v)
        pltpu.sync_copy(dst_hbm.at[wid], dst_v)
        pltpu.sync_copy(asrc_hbm, asrc_v)
        pltpu.sync_copy(adst_hbm, adst_v)

        def zloop(i, carry):
            dacc_v[pl.ds(i * 16, 16)] = jnp.zeros((16,), jnp.float32)
            return carry

        lax.fori_loop(0, _N // 16, zloop, 0)

        def chunk(j, carry):
            def wloop(k, c2):
                sv = src_v[j, pl.ds(k * 16, 16)]
                dv = dst_v[j, pl.ds(k * 16, 16)]
                a = plsc.load_gather(asrc_v, [sv])
                b = plsc.load_gather(adst_v, [dv])
                s = a + b
                w = jnp.exp(jnp.where(s >= 0, s, s * 0.2))
                w_v[j, pl.ds(k * 16, 16)] = w
                plsc.addupdate_scatter(dacc_v, [dv], w)
                return c2

            lax.fori_loop(0, _CH // 16, wloop, 0)
            return carry

        lax.fori_loop(0, _NCH, chunk, 0)

        pltpu.sync_copy(w_v, w_hbm.at[wid])
        pltpu.sync_copy(dacc_v, dacc_hbm.at[pl.ds(wid * _NPAD, _N)])

    return wpass


def _rowpass_kernel_fn():
    mesh = plsc.VectorSubcoreMesh(core_axis_name="c", subcore_axis_name="s")

    @functools.partial(
        pl.kernel,
        out_type=jax.ShapeDtypeStruct((_NC, _NP, _D), jnp.float32),
        mesh=mesh,
        scratch_types=[
            pltpu.VMEM((_GRP, _CH), jnp.int32),
            pltpu.VMEM((_GRP, _CH), jnp.int32),
            pltpu.VMEM((_GRP, _CH), jnp.float32),
            pltpu.VMEM((2, _CH, _D), jnp.float32),
            pltpu.VMEM_SHARED((_NP, _D), jnp.float32),
            pltpu.SemaphoreType.DMA,
            pltpu.SemaphoreType.DMA,
        ],
        compiler_params=pltpu.CompilerParams(needs_layout_passes=False),
    )
    def rowpass(h_hbm, src_hbm, dst_hbm, w_hbm, z_hbm,
                accp_hbm,
                src_v, dst_v, w_v, rowbuf, acc_sh, sem, sem2):
        cid = lax.axis_index("c")
        sid = lax.axis_index("s")
        wid = sid * _NC + cid

        rbase = sid * _STRIPE
        pltpu.sync_copy(z_hbm.at[pl.ds(rbase, _STRIPE)],
                        acc_sh.at[pl.ds(rbase, _STRIPE)])
        plsc.subcore_barrier()

        def group(g, carry):
            pltpu.sync_copy(src_hbm.at[wid, g], src_v)
            pltpu.sync_copy(dst_hbm.at[wid, g], dst_v)
            pltpu.sync_copy(w_hbm.at[wid, g], w_v)

            pltpu.async_copy(h_hbm.at[src_v.at[0]], rowbuf.at[0], sem)

            def chunk(j, c2):
                b = lax.rem(j, 2)
                pltpu.make_async_copy(
                    h_hbm.at[src_v.at[j]], rowbuf.at[b], sem).wait()

                @pl.when((j + 1 < _GRP) & (j > 0))
                def _():
                    pltpu.make_async_copy(
                        rowbuf.at[1 - b], acc_sh.at[dst_v.at[0]],
                        sem2).wait()

                @pl.when(j + 1 < _GRP)
                def _():
                    pltpu.async_copy(
                        h_hbm.at[src_v.at[j + 1]], rowbuf.at[1 - b], sem)

                j16 = jnp.zeros((16,), jnp.int32) + j

                def scale(r, c3):
                    wv = plsc.load_gather(
                        w_v, [j16, jnp.zeros((16,), jnp.int32) + r])
                    for c in range(_D // 16):
                        rowbuf[b, r, pl.ds(c * 16, 16)] = (
                            rowbuf[b, r, pl.ds(c * 16, 16)] * wv)
                    return c3

                lax.fori_loop(0, _CH, scale, 0)
                pltpu.async_copy(rowbuf.at[b], acc_sh.at[dst_v.at[j]],
                                 sem2, add=True)
                return c2

            lax.fori_loop(0, _GRP, chunk, 0)
            pltpu.make_async_copy(
                rowbuf.at[0], acc_sh.at[dst_v.at[0]], sem2).wait()
            pltpu.make_async_copy(
                rowbuf.at[0], acc_sh.at[dst_v.at[0]], sem2).wait()
            return carry

        lax.fori_loop(0, _NG, group, 0)
        plsc.subcore_barrier()

        pltpu.sync_copy(acc_sh.at[pl.ds(rbase, _STRIPE)],
                        accp_hbm.at[cid, pl.ds(rbase, _STRIPE)])

    return rowpass


_wpass = _wpass_kernel_fn()
_rowpass = _rowpass_kernel_fn()


def _edge(h, asrc, adst, src, dst, z):
    w, dacc = _wpass(asrc, adst, src, dst)
    accp = _rowpass(h,
                    src.reshape(_NW, _NG, _GRP, _CH),
                    dst.reshape(_NW, _NG, _GRP, _CH),
                    w.reshape(_NW, _NG, _GRP, _CH), z)
    return accp, dacc


def _full_spec():
    return pl.BlockSpec((_D, _D), lambda i: (0, 0))


def _vec_spec():
    return pl.BlockSpec((_D,), lambda i: (0,))


def _row_spec():
    return pl.BlockSpec((_BLK, _D), lambda i: (i, 0))


def _n_spec():
    return pl.BlockSpec((_NPAD,), lambda i: (0,))


def _prep1(x, Wp, bp, W1, as1, ad1):
    def body(x_r, Wp_r, bp_r, W1_r, as1_r, ad1_r, h1_r, s_r, d_r):
        i = pl.program_id(0)
        h0 = jnp.dot(x_r[...], Wp_r[...],
                     preferred_element_type=jnp.float32) + bp_r[...][None, :]
        h1 = jnp.dot(h0, W1_r[...], preferred_element_type=jnp.float32)
        h1_r[...] = h1
        s_r[pl.ds(i * _BLK, _BLK)] = jnp.sum(h1 * as1_r[...][None, :], axis=1)
        d_r[pl.ds(i * _BLK, _BLK)] = jnp.sum(h1 * ad1_r[...][None, :], axis=1)

    return pl.pallas_call(
        body,
        grid=(_NB,),
        in_specs=[_row_spec(), _full_spec(), _vec_spec(), _full_spec(),
                  _vec_spec(), _vec_spec()],
        out_specs=[_row_spec(), _n_spec(), _n_spec()],
        out_shape=[
            jax.ShapeDtypeStruct((_N, _D), jnp.float32),
            jax.ShapeDtypeStruct((_NPAD,), jnp.float32),
            jax.ShapeDtypeStruct((_NPAD,), jnp.float32),
        ],
    )(x, Wp, bp, W1, as1, ad1)


def _dacc_spec():
    return pl.BlockSpec((_NW, _NPAD), lambda i: (0, 0))


def _combine_mid(acc0, acc1, dacc, asrc, adst, h1, b1, W2, as2, ad2):
    def body(a0, a1, dc, sr, dr, h1r, b1r, W2r, as2r, ad2r,
             h2_r, s2_r, d2_r):
        i = pl.program_id(0)
        blk = pl.ds(i * _BLK, _BLK)
        s = sr[blk] + dr[blk]
        wl = jnp.exp(jnp.where(s >= 0, s, s * 0.2))
        num = a0[...] + a1[...] + wl[:, None] * h1r[...]
        den = jnp.sum(dc[:, blk], axis=0) + wl + 1e-16
        g = jnp.maximum(num / den[:, None] + b1r[...][None, :], 0.0)
        h2 = jnp.dot(g, W2r[...], preferred_element_type=jnp.float32)
        h2_r[...] = h2
        s2_r[blk] = jnp.sum(h2 * as2r[...][None, :], axis=1)
        d2_r[blk] = jnp.sum(h2 * ad2r[...][None, :], axis=1)

    return pl.pallas_call(
        body,
        grid=(_NB,),
        in_specs=[_row_spec(), _row_spec(), _dacc_spec(), _n_spec(),
                  _n_spec(), _row_spec(), _vec_spec(), _full_spec(),
                  _vec_spec(), _vec_spec()],
        out_specs=[_row_spec(), _n_spec(), _n_spec()],
        out_shape=[
            jax.ShapeDtypeStruct((_N, _D), jnp.float32),
            jax.ShapeDtypeStruct((_NPAD,), jnp.float32),
            jax.ShapeDtypeStruct((_NPAD,), jnp.float32),
        ],
    )(acc0, acc1, dacc, asrc, adst, h1, b1, W2, as2, ad2)


def _combine_final(acc0, acc1, dacc, asrc, adst, h2, b2):
    def body(a0, a1, dc, sr, dr, h2r, b2r, out_r):
        i = pl.program_id(0)
        blk = pl.ds(i * _BLK, _BLK)
        s = sr[blk] + dr[blk]
        wl = jnp.exp(jnp.where(s >= 0, s, s * 0.2))
        num = a0[...] + a1[...] + wl[:, None] * h2r[...]
        den = jnp.sum(dc[:, blk], axis=0) + wl + 1e-16
        out_r[...] = num / den[:, None] + b2r[...][None, :]

    return pl.pallas_call(
        body,
        grid=(_NB,),
        in_specs=[_row_spec(), _row_spec(), _dacc_spec(), _n_spec(),
                  _n_spec(), _row_spec(), _vec_spec()],
        out_specs=_row_spec(),
        out_shape=jax.ShapeDtypeStruct((_N, _D), jnp.float32),
    )(acc0, acc1, dacc, asrc, adst, h2, b2)


def kernel(x, edge_index, Wp, bp, W1, as1, ad1, b1, W2, as2, ad2, b2):
    src = edge_index[0].reshape(_NW, _NCH, _CH)
    dst = edge_index[1].reshape(_NW, _NCH, _CH)
    z = jnp.zeros((_NP, _D), jnp.float32)

    h1, s1, d1 = _prep1(x, Wp, bp, W1, as1, ad1)
    accp1, dacc1 = _edge(h1, s1[:_N], d1[:_N], src, dst, z)
    h2, s2, d2 = _combine_mid(accp1[0], accp1[1], dacc1.reshape(_NW, _NPAD),
                              s1, d1, h1, b1, W2, as2, ad2)
    accp2, dacc2 = _edge(h2, s2[:_N], d2[:_N], src, dst, z)
    return _combine_final(accp2[0], accp2[1], dacc2.reshape(_NW, _NPAD),
                          s2, d2, h2, b2)
